# baseline (speedup 1.0000x reference)
import functools

import jax
import jax.numpy as jnp
from jax.experimental import pallas as pl

T = 16384
INPUT_DIM = 256
K = 8192
D = 32
NUM_Q = 8
TB = 128


def _bf16_round(v):
    u = jax.lax.bitcast_convert_type(v, jnp.uint32)
    bias = jnp.uint32(0x7FFF) + ((u >> 16) & jnp.uint32(1))
    u2 = (u + bias) & jnp.uint32(0xFFFF0000)
    return jax.lax.bitcast_convert_type(u2, jnp.float32)


def _rvq_body(x_ref, w_in_ref, w_out_ref, emb_ref, out_ref, codes_ref):
    f32 = jnp.float32
    x = x_ref[...]
    x_proj = jax.lax.dot_general(
        _bf16_round(x), _bf16_round(w_in_ref[...]), (((1,), (1,)), ((), ())),
        preferred_element_type=f32,
    )
    residual = x_proj
    iota = jax.lax.broadcasted_iota(jnp.int32, (TB, K), 1)
    for q in range(NUM_Q):
        e = emb_ref[q]
        eq = _bf16_round(e)
        e_sq = jnp.sum(e * e, axis=1)
        x_sq = jnp.sum(residual * residual, axis=1, keepdims=True)
        cross = jax.lax.dot_general(
            residual, eq, (((1,), (1,)), ((), ())),
            preferred_element_type=f32,
        )
        dist = x_sq - 2.0 * cross + e_sq[None, :]
        min_d = jnp.min(dist, axis=1, keepdims=True)
        idx = jnp.min(jnp.where(dist == min_d, iota, K), axis=1)
        codes_ref[q, :] = idx
        onehot = (iota == idx[:, None]).astype(f32)
        quant = jax.lax.dot_general(
            onehot, e, (((1,), (0,)), ((), ())), preferred_element_type=f32
        )
        residual = residual - quant
    out_sum = x_proj - residual
    out_ref[...] = jax.lax.dot_general(
        out_sum, w_out_ref[...], (((1,), (1,)), ((), ())),
        preferred_element_type=f32,
    )


@functools.partial(jax.jit, static_argnames=("interpret",))
def kernel(x_td, w_in_oik, w_out_oik, embeddings_qkd, interpret=False):
    w_in = w_in_oik[:, :, 0]
    w_out = w_out_oik[:, :, 0]
    out_td, codes_qt = pl.pallas_call(
        _rvq_body,
        grid=(T // TB,),
        in_specs=[
            pl.BlockSpec((TB, INPUT_DIM), lambda i: (i, 0)),
            pl.BlockSpec((D, INPUT_DIM), lambda i: (0, 0)),
            pl.BlockSpec((INPUT_DIM, D), lambda i: (0, 0)),
            pl.BlockSpec((NUM_Q, K, D), lambda i: (0, 0, 0)),
        ],
        out_specs=[
            pl.BlockSpec((TB, INPUT_DIM), lambda i: (i, 0)),
            pl.BlockSpec((NUM_Q, TB), lambda i: (0, i)),
        ],
        out_shape=[
            jax.ShapeDtypeStruct((T, INPUT_DIM), jnp.float32),
            jax.ShapeDtypeStruct((NUM_Q, T), jnp.int32),
        ],
        interpret=interpret,
    )(x_td, w_in, w_out, embeddings_qkd)
    return (out_td, codes_qt)
